# Initial kernel scaffold; baseline (speedup 1.0000x reference)
import jax
import jax.numpy as jnp
from jax.experimental import pallas as pl


def kernel(x):
    raise NotImplementedError("write your pallas kernel here")


# speedup vs baseline: 7.7691x; 7.7691x over previous
import jax
import jax.numpy as jnp
from jax.experimental import pallas as pl
from jax.experimental.pallas import tpu as pltpu

_LOW, _HIGH = 0.1, 0.3
_CH = 3
_T1 = 0.41421317
_T2 = 2.41420677


def _canny_kernel(x_ref, o_ref):
    x = x_ref[0]
    H, L = x.shape
    z_row = jnp.zeros((1, L), jnp.float32)
    z_col = jnp.zeros((H, _CH), jnp.float32)

    def up(t):
        return jnp.concatenate([z_row, t[:-1, :]], axis=0)

    def dn(t):
        return jnp.concatenate([t[1:, :], z_row], axis=0)

    def lf(t):
        return jnp.concatenate([z_col, t[:, :-_CH]], axis=1)

    def rt(t):
        return jnp.concatenate([t[:, _CH:], z_col], axis=1)

    def wup(t):
        return jnp.roll(t, 1, axis=0)

    def wdn(t):
        return jnp.roll(t, -1, axis=0)

    def wlf(t):
        return jnp.roll(t, _CH, axis=1)

    def wrt(t):
        return jnp.roll(t, -_CH, axis=1)

    def nbr(t, dh, dw):
        if dh == -1:
            t = up(t)
        elif dh == 1:
            t = dn(t)
        if dw == -1:
            t = lf(t)
        elif dw == 1:
            t = rt(t)
        return t

    def conv3(t, weights):
        acc = None
        for (dh, dw, w) in weights:
            term = nbr(t, dh, dw) * w
            acc = term if acc is None else acc + term
        return acc

    xb = x.astype(jnp.bfloat16).astype(jnp.float32)
    g1, g2, g4 = 0.0625, 0.125, 0.25
    blurred = conv3(xb, [
        (-1, -1, g1), (-1, 0, g2), (-1, 1, g1),
        (0, -1, g2), (0, 0, g4), (0, 1, g2),
        (1, -1, g1), (1, 0, g2), (1, 1, g1),
    ])
    blurred = blurred.astype(jnp.bfloat16).astype(jnp.float32)
    gx = conv3(blurred, [
        (-1, -1, -1.0), (-1, 1, 1.0),
        (0, -1, -2.0), (0, 1, 2.0),
        (1, -1, -1.0), (1, 1, 1.0),
    ])
    gy = conv3(blurred, [
        (-1, -1, -1.0), (-1, 0, -2.0), (-1, 1, -1.0),
        (1, -1, 1.0), (1, 0, 2.0), (1, 1, 1.0),
    ])

    mag = jnp.sqrt(gx * gx + gy * gy)

    ax = jnp.abs(gx)
    ay = jnp.abs(gy)
    b0 = ay <= _T1 * ax
    b90 = ay > _T2 * ax
    diag = jnp.logical_not(b0) & jnp.logical_not(b90)
    pos = (gx * gy) > 0.0
    b45 = diag & pos
    b135 = diag & jnp.logical_not(pos)

    ml = wlf(mag)
    mr = wrt(mag)
    mu = wup(mag)
    md = wdn(mag)
    keep = (
        (b0 & (mag >= mr) & (mag >= ml))
        | (b45 & (mag >= wdn(ml)) & (mag >= wup(mr)))
        | (b90 & (mag >= md) & (mag >= mu))
        | (b135 & (mag >= wdn(mr)) & (mag >= wup(ml)))
    )
    sup = jnp.where(keep, mag, 0.0)

    strong = jnp.where(sup >= _HIGH, 1.0, 0.0)
    weak = (sup >= _LOW) & (sup < _HIGH)

    def cond(carry):
        return carry[1]

    def body(carry):
        e, _ = carry
        rows = wlf(e) + e + wrt(e)
        box = wup(rows) + rows + wdn(rows)
        new = jnp.where(weak & (box > 0.0), 1.0, e)
        return new, jnp.any(new != e)

    edges, _ = jax.lax.while_loop(cond, body, (strong, jnp.bool_(True)))
    o_ref[0] = edges


def kernel(x):
    B, H, W, C = x.shape
    xr = x.reshape(B, H, W * C)
    out = pl.pallas_call(
        _canny_kernel,
        grid=(B,),
        in_specs=[pl.BlockSpec((1, H, W * C), lambda i: (i, 0, 0))],
        out_specs=pl.BlockSpec((1, H, W * C), lambda i: (i, 0, 0)),
        out_shape=jax.ShapeDtypeStruct((B, H, W * C), jnp.float32),
        compiler_params=pltpu.CompilerParams(
            dimension_semantics=("parallel",),
            vmem_limit_bytes=100 * 1024 * 1024,
        ),
    )(xr)
    return out.reshape(B, H, W, C)


# speedup vs baseline: 9.0574x; 1.1658x over previous
import jax
import jax.numpy as jnp
from jax.experimental import pallas as pl
from jax.experimental.pallas import tpu as pltpu

_LOW, _HIGH = 0.1, 0.3
_CH = 3
_T1 = 0.41421317
_T2 = 2.41420677


def _canny_kernel(x_ref, o_ref):
    x = x_ref[0]
    H, L = x.shape
    z_row = jnp.zeros((1, L), jnp.float32)
    z_col = jnp.zeros((H, _CH), jnp.float32)

    def up(t):
        return jnp.concatenate([z_row, t[:-1, :]], axis=0)

    def dn(t):
        return jnp.concatenate([t[1:, :], z_row], axis=0)

    def lf(t):
        return jnp.concatenate([z_col, t[:, :-_CH]], axis=1)

    def rt(t):
        return jnp.concatenate([t[:, _CH:], z_col], axis=1)

    def wup(t):
        return jnp.roll(t, 1, axis=0)

    def wdn(t):
        return jnp.roll(t, -1, axis=0)

    def wlf(t):
        return jnp.roll(t, _CH, axis=1)

    def wrt(t):
        return jnp.roll(t, -_CH, axis=1)

    def nbr(t, dh, dw):
        if dh == -1:
            t = up(t)
        elif dh == 1:
            t = dn(t)
        if dw == -1:
            t = lf(t)
        elif dw == 1:
            t = rt(t)
        return t

    def conv3(t, weights):
        acc = None
        for (dh, dw, w) in weights:
            term = nbr(t, dh, dw) * w
            acc = term if acc is None else acc + term
        return acc

    xb = x.astype(jnp.bfloat16).astype(jnp.float32)
    g1, g2, g4 = 0.0625, 0.125, 0.25
    blurred = conv3(xb, [
        (-1, -1, g1), (-1, 0, g2), (-1, 1, g1),
        (0, -1, g2), (0, 0, g4), (0, 1, g2),
        (1, -1, g1), (1, 0, g2), (1, 1, g1),
    ])
    blurred = blurred.astype(jnp.bfloat16).astype(jnp.float32)
    gx = conv3(blurred, [
        (-1, -1, -1.0), (-1, 1, 1.0),
        (0, -1, -2.0), (0, 1, 2.0),
        (1, -1, -1.0), (1, 1, 1.0),
    ])
    gy = conv3(blurred, [
        (-1, -1, -1.0), (-1, 0, -2.0), (-1, 1, -1.0),
        (1, -1, 1.0), (1, 0, 2.0), (1, 1, 1.0),
    ])

    mag = jnp.sqrt(gx * gx + gy * gy)

    ax = jnp.abs(gx)
    ay = jnp.abs(gy)
    b0 = ay <= _T1 * ax
    b90 = ay > _T2 * ax
    pos = (gx * gy) > 0.0

    ml = wlf(mag)
    mr = wrt(mag)
    mu = wup(mag)
    md = wdn(mag)
    n1 = jnp.where(b0, mr, jnp.where(b90, md, jnp.where(pos, wdn(ml), wdn(mr))))
    n2 = jnp.where(b0, ml, jnp.where(b90, mu, jnp.where(pos, wup(mr), wup(ml))))
    keep = (mag >= n1) & (mag >= n2)
    sup = jnp.where(keep, mag, 0.0)

    strong = jnp.where(sup >= _HIGH, 1.0, 0.0)
    weak = (sup >= _LOW) & (sup < _HIGH)

    def step(e):
        rows = wlf(e) + e + wrt(e)
        box = wup(rows) + rows + wdn(rows)
        return jnp.where(weak & (box > 0.0), 1.0, e)

    def cond(carry):
        return carry[1]

    def body(carry):
        e, _ = carry
        e1 = step(carry[0])
        e2 = step(e1)
        return e2, jnp.any(e2 != e1)

    edges, _ = jax.lax.while_loop(cond, body, (strong, jnp.bool_(True)))
    o_ref[0] = edges


def kernel(x):
    B, H, W, C = x.shape
    xr = x.reshape(B, H, W * C)
    out = pl.pallas_call(
        _canny_kernel,
        grid=(B,),
        in_specs=[pl.BlockSpec((1, H, W * C), lambda i: (i, 0, 0))],
        out_specs=pl.BlockSpec((1, H, W * C), lambda i: (i, 0, 0)),
        out_shape=jax.ShapeDtypeStruct((B, H, W * C), jnp.float32),
        compiler_params=pltpu.CompilerParams(
            dimension_semantics=("parallel",),
            vmem_limit_bytes=100 * 1024 * 1024,
        ),
    )(xr)
    return out.reshape(B, H, W, C)
